# baseline (speedup 1.0000x reference)
import functools

import jax
import jax.numpy as jnp
from jax import lax
from jax.experimental import pallas as pl
from jax.experimental.pallas import tpu as pltpu
from jax.experimental.pallas import tpu_sc as plsc

N_NODES = 10000
D = 128
N_EDGES = 320000

NC = 2
NS = 16
EPT = N_EDGES // NS
EPT_PAD = 20480
CHUNK = 128
NCHUNK = EPT_PAD // CHUNK
N_HALF = N_NODES // NC
N_LOC = 5120
ROWS_PER_TILE = N_LOC // NS
CNT_W = 16


def _make_seg_sum(with_cnt):
  mesh = plsc.VectorSubcoreMesh(core_axis_name="c", subcore_axis_name="s")
  out_type = [jax.ShapeDtypeStruct((NC, N_LOC, D), jnp.float32)]
  scratch = [
      pltpu.VMEM((CHUNK,), jnp.int32),
      pltpu.VMEM((CHUNK,), jnp.int32),
      pltpu.VMEM((CHUNK, D), jnp.float32),
      pltpu.VMEM_SHARED((N_LOC, D), jnp.float32),
      pltpu.VMEM_SHARED((NS, 2, CHUNK), jnp.int32),
      pltpu.SemaphoreType.DMA,
  ]
  if with_cnt:
    out_type.append(jax.ShapeDtypeStruct((NC, N_LOC, CNT_W), jnp.float32))
    scratch += [
        pltpu.VMEM((CHUNK, CNT_W), jnp.float32),
        pltpu.VMEM_SHARED((N_LOC, CNT_W), jnp.float32),
    ]

  def body(x_hbm, src_hbm, dst_hbm, *rest):
    if with_cnt:
      (out_hbm, cnt_hbm, sidx, didx, gbuf, acc_sh, idx_sp, sem,
       onesb, cnt_sh) = rest
    else:
      out_hbm, sidx, didx, gbuf, acc_sh, idx_sp, sem = rest
    c = lax.axis_index("c")
    s = lax.axis_index("s")
    row0 = s * ROWS_PER_TILE

    zero16 = jnp.zeros((16,), jnp.float32)
    one16 = jnp.ones((16,), jnp.float32)

    def zrow(i, _):
      def zcol(j, _):
        gbuf[i, pl.ds(j * 16, 16)] = zero16
        return 0
      return lax.fori_loop(0, D // 16, zcol, 0)
    lax.fori_loop(0, CHUNK, zrow, 0)

    pltpu.sync_copy(gbuf, acc_sh.at[pl.ds(row0, CHUNK)])
    pltpu.sync_copy(gbuf, acc_sh.at[pl.ds(row0 + CHUNK, CHUNK)])
    pltpu.sync_copy(gbuf.at[pl.ds(0, 64)],
                    acc_sh.at[pl.ds(row0 + 2 * CHUNK, 64)])

    if with_cnt:
      def zorow(i, _):
        onesb[i, :] = zero16
        return 0
      lax.fori_loop(0, CHUNK, zorow, 0)
      pltpu.sync_copy(onesb, cnt_sh.at[pl.ds(row0, CHUNK)])
      pltpu.sync_copy(onesb, cnt_sh.at[pl.ds(row0 + CHUNK, CHUNK)])
      pltpu.sync_copy(onesb.at[pl.ds(0, 64)],
                      cnt_sh.at[pl.ds(row0 + 2 * CHUNK, 64)])

      def orow(i, _):
        onesb[i, :] = one16
        return 0
      lax.fori_loop(0, CHUNK, orow, 0)

    lo = c * N_HALF
    lov = jnp.full((16,), lo, jnp.int32)
    zv = jnp.zeros((16,), jnp.int32)
    nhv = jnp.full((16,), N_HALF, jnp.int32)
    trashv = jnp.full((16,), N_HALF + s, jnp.int32)

    plsc.subcore_barrier()

    def step(j, _):
      o = j * CHUNK
      pltpu.sync_copy(src_hbm.at[pl.ds(s * EPT_PAD + o, CHUNK)],
                      idx_sp.at[s, 0])
      pltpu.sync_copy(dst_hbm.at[pl.ds(s * EPT_PAD + o, CHUNK)],
                      idx_sp.at[s, 1])
      pltpu.sync_copy(idx_sp.at[s, 0], sidx)
      pltpu.sync_copy(idx_sp.at[s, 1], didx)

      def cstep(i, _):
        oo = i * 16
        d = didx[pl.ds(oo, 16)]
        dl = d - lov
        m = (dl >= zv) & (dl < nhv)
        didx[pl.ds(oo, 16)] = jnp.where(m, dl, trashv)
        sv = sidx[pl.ds(oo, 16)]
        sidx[pl.ds(oo, 16)] = jnp.where(m, sv, zv)
        return 0
      lax.fori_loop(0, CHUNK // 16, cstep, 0)

      pltpu.sync_copy(x_hbm.at[sidx], gbuf)
      pltpu.sync_copy(gbuf, acc_sh.at[didx], add=True)
      if with_cnt:
        pltpu.sync_copy(onesb, cnt_sh.at[didx], add=True)
      return 0
    lax.fori_loop(0, NCHUNK, step, 0)

    plsc.subcore_barrier()

    pltpu.sync_copy(acc_sh.at[pl.ds(row0, ROWS_PER_TILE)],
                    out_hbm.at[c, pl.ds(row0, ROWS_PER_TILE)])
    if with_cnt:
      pltpu.sync_copy(cnt_sh.at[pl.ds(row0, ROWS_PER_TILE)],
                      cnt_hbm.at[c, pl.ds(row0, ROWS_PER_TILE)])

  return pl.kernel(body, mesh=mesh, out_type=out_type, scratch_types=scratch)


_seg_sum_cnt = _make_seg_sum(True)

_BLK = 1000
_BPC = N_HALF // _BLK


def _tc_layer_body(p_ref, c_ref, x_ref, wl_ref, b_ref, wr_ref, lb_ref,
                   o_ref):
  cnt = jnp.maximum(c_ref[0, :, 0:1], 1.0)
  mean = p_ref[0] / cnt
  dn = (((1,), (1,)), ((), ()))
  acc = lax.dot_general(mean, wl_ref[...], dn,
                        preferred_element_type=jnp.float32)
  acc = acc + lax.dot_general(x_ref[...], wr_ref[...], dn,
                              preferred_element_type=jnp.float32)
  acc = acc + b_ref[...]
  o_ref[...] = jnp.maximum(acc, lb_ref[...])


def _tc_layer(parts, cnts, x, W_l, b_l, W_r, lb):
  grid = (N_NODES // _BLK,)
  return pl.pallas_call(
      _tc_layer_body,
      grid=grid,
      in_specs=(
          pl.BlockSpec((1, _BLK, D), lambda i: (i // _BPC, i % _BPC, 0)),
          pl.BlockSpec((1, _BLK, CNT_W), lambda i: (i // _BPC, i % _BPC, 0)),
          pl.BlockSpec((_BLK, D), lambda i: (i, 0)),
          pl.BlockSpec((D, D), lambda i: (0, 0)),
          pl.BlockSpec((1, D), lambda i: (0, 0)),
          pl.BlockSpec((D, D), lambda i: (0, 0)),
          pl.BlockSpec((1, D), lambda i: (0, 0)),
      ),
      out_specs=pl.BlockSpec((_BLK, D), lambda i: (i, 0)),
      out_shape=jax.ShapeDtypeStruct((N_NODES, D), jnp.float32),
  )(parts, cnts, x, W_l, b_l, W_r, lb)


def kernel(x, edge_index, W1_l, b1_l, W1_r, W2_l, b2_l, W2_r):
  ei = edge_index.astype(jnp.int32)
  pad = jnp.zeros((NS, EPT_PAD - EPT), jnp.int32)
  src = jnp.concatenate([ei[0].reshape(NS, EPT), pad], axis=1)
  src = src.reshape(NS * EPT_PAD)
  dst = jnp.concatenate([ei[1].reshape(NS, EPT), pad + N_NODES], axis=1)
  dst = dst.reshape(NS * EPT_PAD)

  Wls = jnp.stack([W1_l, W2_l])
  Wrs = jnp.stack([W1_r, W2_r])
  bs = jnp.stack([b1_l.reshape(1, D), b2_l.reshape(1, D)])
  lbs = jnp.stack([jnp.zeros((1, D), jnp.float32),
                   jnp.full((1, D), -jnp.inf, jnp.float32)])

  def layer(h, ops):
    wl, wr, b, lb = ops
    parts, cnts = _seg_sum_cnt(h, src, dst)
    h2 = _tc_layer(parts, cnts, h, wl, b, wr, lb)
    return h2, None

  out, _ = lax.scan(layer, x, (Wls, Wrs, bs, lbs))
  return out
